# baseline (speedup 1.0000x reference)
import functools
import math

import jax
import jax.numpy as jnp
from jax import lax
from jax.experimental import pallas as pl
from jax.experimental.pallas import tpu as pltpu
from jax.experimental.pallas import tpu_sc as plsc

N = 10000
E = 320000
INDIM = 128
DIM1 = 32
NGRAPH = 16
N_PER = N // NGRAPH
K = int(math.ceil(0.5 * N_PER))

NC = 2
NS = 16
NW = NC * NS
EW = E // NW
NLOC = 320
NPAD = NW * NLOC
CAP = 16384
NCH = CAP // 128

_BLK1 = 512
_G1 = (N + _BLK1 - 1) // _BLK1
_BLKE = 4096
_GE = (E + _BLKE - 1) // _BLKE


def _k1_body(x_ref, w_ref, as_v, ad_v, h_ref, as_ref, ad_ref):
    h = jnp.dot(x_ref[...], w_ref[...], preferred_element_type=jnp.float32)
    h_ref[...] = h
    as_ref[...] = jnp.dot(h, as_v[...], preferred_element_type=jnp.float32)
    ad_ref[...] = jnp.dot(h, ad_v[...], preferred_element_type=jnp.float32)


def _phase1(x, W_gat, att_src, att_dst):
    return pl.pallas_call(
        _k1_body,
        grid=(_G1,),
        in_specs=[
            pl.BlockSpec((_BLK1, INDIM), lambda i: (i, 0)),
            pl.BlockSpec((INDIM, DIM1), lambda i: (0, 0)),
            pl.BlockSpec((DIM1, 1), lambda i: (0, 0)),
            pl.BlockSpec((DIM1, 1), lambda i: (0, 0)),
        ],
        out_specs=[
            pl.BlockSpec((_BLK1, DIM1), lambda i: (i, 0)),
            pl.BlockSpec((_BLK1, 1), lambda i: (i, 0)),
            pl.BlockSpec((_BLK1, 1), lambda i: (i, 0)),
        ],
        out_shape=[
            jax.ShapeDtypeStruct((N, DIM1), jnp.float32),
            jax.ShapeDtypeStruct((N, 1), jnp.float32),
            jax.ShapeDtypeStruct((N, 1), jnp.float32),
        ],
    )(x, W_gat, att_src.reshape(DIM1, 1), att_dst.reshape(DIM1, 1))


def _kae_body(ea_ref, le_ref, ate_ref, o_ref):
    t = ea_ref[...] * le_ref[...]
    o_ref[...] = jnp.dot(t, ate_ref[...], preferred_element_type=jnp.float32)


def _phase_ae(edge_attr, lin_edge, att_edge):
    return pl.pallas_call(
        _kae_body,
        grid=(_GE,),
        in_specs=[
            pl.BlockSpec((_BLKE, 1), lambda i: (i, 0)),
            pl.BlockSpec((1, DIM1), lambda i: (0, 0)),
            pl.BlockSpec((DIM1, 1), lambda i: (0, 0)),
        ],
        out_specs=pl.BlockSpec((_BLKE, 1), lambda i: (i, 0)),
        out_shape=jax.ShapeDtypeStruct((E, 1), jnp.float32),
    )(edge_attr, lin_edge, att_edge.reshape(DIM1, 1))


_mesh = plsc.VectorSubcoreMesh(core_axis_name="c", subcore_axis_name="s")

_LANE0 = None


def _splat_i(i):
    return jnp.broadcast_to(i, (16,)).astype(jnp.int32)


def _sload(ref, i):
    return plsc.load_gather(ref, [_splat_i(i)])[0]


def _sstore(ref, i, v, mask0):
    plsc.store_scatter(ref, [_splat_i(i)], jnp.broadcast_to(v, (16,)), mask=mask0)


@functools.partial(
    pl.kernel, mesh=_mesh,
    compiler_params=pltpu.CompilerParams(needs_layout_passes=False),
    out_type=[
        jax.ShapeDtypeStruct((E,), jnp.float32),
        jax.ShapeDtypeStruct((NW, CAP), jnp.int32),
        jax.ShapeDtypeStruct((NW, 16), jnp.int32),
    ],
    scratch_types=[
        pltpu.VMEM((10048,), jnp.float32),
        pltpu.VMEM((10048,), jnp.float32),
        pltpu.VMEM((EW,), jnp.int32),
        pltpu.VMEM((EW,), jnp.int32),
        pltpu.VMEM((EW,), jnp.float32),
        pltpu.VMEM((EW,), jnp.float32),
        pltpu.VMEM((2000,), jnp.int32),
        pltpu.VMEM((CAP,), jnp.int32),
        pltpu.VMEM((16,), jnp.int32),
    ],
)
def _sc2(src_hbm, dst_hbm, ae_hbm, as_hbm, ad_hbm,
         e_hbm, elist_hbm, cnt_hbm,
         asv, adv, sv, dv, aev, ev, dbuf, elv, cntv):
    wid = lax.axis_index("s") * NC + lax.axis_index("c")
    base = wid * EW
    pltpu.sync_copy(as_hbm, asv.at[pl.ds(0, N)])
    pltpu.sync_copy(ad_hbm, adv.at[pl.ds(0, N)])
    pltpu.sync_copy(src_hbm.at[pl.ds(base, EW)], sv)
    pltpu.sync_copy(dst_hbm.at[pl.ds(base, EW)], dv)
    pltpu.sync_copy(ae_hbm.at[pl.ds(base, EW)], aev)

    def ebody(j, _):
        s16 = sv[pl.ds(j * 16, 16)]
        d16 = dv[pl.ds(j * 16, 16)]
        a1 = plsc.load_gather(asv, [s16])
        a2 = plsc.load_gather(adv, [d16])
        t = (a1 + a2) + aev[pl.ds(j * 16, 16)]
        ev[pl.ds(j * 16, 16)] = jnp.where(t >= 0, t, 0.2 * t)
        return 0

    lax.fori_loop(0, EW // 16, ebody, 0)
    pltpu.sync_copy(ev, e_hbm.at[pl.ds(base, EW)])

    z16 = jnp.zeros((16,), jnp.int32)

    def zbody(j, _):
        elv[pl.ds(j * 16, 16)] = z16
        return 0

    lax.fori_loop(0, CAP // 16, zbody, 0)

    n0 = wid * NLOC
    n1 = n0 + NLOC
    iota = lax.iota(jnp.int32, 16)

    def cbody(ch, wo):
        pltpu.sync_copy(dst_hbm.at[pl.ds(ch * 2000, 2000)], dbuf)

        def ibody(k, wo2):
            d16 = dbuf[pl.ds(k * 16, 16)]
            ids = (ch * 2000 + k * 16) + iota
            msk = (d16 >= n0) & (d16 < n1)
            plsc.store_compressed(elv.at[pl.ds(wo2, 16)], ids, mask=msk)
            pc = plsc.all_reduce_population_count(msk)
            return wo2 + pc[0]

        return lax.fori_loop(0, 125, ibody, wo)

    wo = lax.fori_loop(0, E // 2000, cbody, 0)
    cntv[...] = jnp.broadcast_to(wo, (16,)).astype(jnp.int32)
    pltpu.sync_copy(elv, elist_hbm.at[wid])
    pltpu.sync_copy(cntv, cnt_hbm.at[wid])


@functools.partial(
    pl.kernel, mesh=_mesh,
    compiler_params=pltpu.CompilerParams(needs_layout_passes=False),
    out_type=jax.ShapeDtypeStruct((NPAD, DIM1), jnp.float32),
    scratch_types=[
        pltpu.VMEM((CAP,), jnp.int32),
        pltpu.VMEM((CAP,), jnp.int32),
        pltpu.VMEM((CAP,), jnp.int32),
        pltpu.VMEM((CAP,), jnp.float32),
        pltpu.VMEM((384,), jnp.float32),
        pltpu.VMEM((384,), jnp.float32),
        pltpu.VMEM((NLOC, DIM1), jnp.float32),
        pltpu.VMEM((128, 128), jnp.float32),
        pltpu.VMEM((16,), jnp.int32),
        pltpu.VMEM((DIM1,), jnp.float32),
        pltpu.SemaphoreType.DMA,
    ],
)
def _sc3(src_hbm, dst_hbm, e_hbm, h_hbm, elist_hbm, cnt_hbm, bias_hbm,
         hgat_hbm,
         elv, ldst, lsrc, lex, em, den, acc, rows, cntv, biasv, sem):
    wid = lax.axis_index("s") * NC + lax.axis_index("c")
    n0 = wid * NLOC
    pltpu.sync_copy(elist_hbm.at[wid], elv)
    pltpu.sync_copy(cnt_hbm.at[wid], cntv)
    pltpu.sync_copy(bias_hbm, biasv)
    m = cntv[pl.ds(0, 16)][0]
    iota = lax.iota(jnp.int32, 16)
    mask0 = iota == 0

    hs = []
    for ci in range(NCH):
        sl = pl.ds(ci * 128, 128)
        hs.append(pltpu.async_copy(dst_hbm.at[elv.at[sl]], ldst.at[sl], sem))
    for h in hs:
        h.wait()
    hs = []
    for ci in range(NCH):
        sl = pl.ds(ci * 128, 128)
        hs.append(pltpu.async_copy(src_hbm.at[elv.at[sl]], lsrc.at[sl], sem))
    for h in hs:
        h.wait()
    hs = []
    for ci in range(NCH):
        sl = pl.ds(ci * 128, 128)
        hs.append(pltpu.async_copy(e_hbm.at[elv.at[sl]], lex.at[sl], sem))
    for h in hs:
        h.wait()

    ninf16 = jnp.full((16,), -jnp.inf, jnp.float32)
    zf16 = jnp.zeros((16,), jnp.float32)

    def initem(j, _):
        em[pl.ds(j * 16, 16)] = ninf16
        den[pl.ds(j * 16, 16)] = zf16
        return 0

    lax.fori_loop(0, 24, initem, 0)

    def emaxb(i, _):
        d = _sload(ldst, i) - n0
        ei = _sload(lex, i)
        _sstore(em, d, jnp.maximum(_sload(em, d), ei), mask0)
        return 0

    lax.fori_loop(0, m, emaxb, 0)

    def emfin(j, _):
        v = em[pl.ds(j * 16, 16)]
        em[pl.ds(j * 16, 16)] = jnp.where(v == -jnp.inf, 0.0, v)
        return 0

    lax.fori_loop(0, 24, emfin, 0)

    def exb(j, _):
        dl = jnp.clip(ldst[pl.ds(j * 16, 16)] - n0, 0, 383)
        eg = plsc.load_gather(em, [dl])
        lex[pl.ds(j * 16, 16)] = jnp.exp(lex[pl.ds(j * 16, 16)] - eg)
        return 0

    lax.fori_loop(0, CAP // 16, exb, 0)

    def denb(i, _):
        d = _sload(ldst, i) - n0
        _sstore(den, d, _sload(den, d) + _sload(lex, i), mask0)
        return 0

    lax.fori_loop(0, m, denb, 0)

    def dene(j, _):
        den[pl.ds(j * 16, 16)] = den[pl.ds(j * 16, 16)] + 1e-16
        return 0

    lax.fori_loop(0, 24, dene, 0)

    def alb(j, _):
        dl = jnp.clip(ldst[pl.ds(j * 16, 16)] - n0, 0, 383)
        dg = plsc.load_gather(den, [dl])
        lex[pl.ds(j * 16, 16)] = lex[pl.ds(j * 16, 16)] / dg
        return 0

    lax.fori_loop(0, CAP // 16, alb, 0)

    iota16 = iota + 16

    def zacc(r, _):
        plsc.store_scatter(acc, [_splat_i(r), iota], zf16)
        plsc.store_scatter(acc, [_splat_i(r), iota16], zf16)
        return 0

    lax.fori_loop(0, NLOC, zacc, 0)

    def chunk(ci, _):
        sl = pl.ds(ci * 128, 128)
        pltpu.async_copy(h_hbm.at[lsrc.at[sl]], rows, sem).wait()
        nb = jnp.clip(m - ci * 128, 0, 128)

        def rb(r, _2):
            gi = ci * 128 + r
            al = _sload(lex, gi)
            d = _sload(ldst, gi) - n0
            rs = _splat_i(r)
            dsp = _splat_i(d)
            r0 = plsc.load_gather(rows, [rs, iota]) * al
            r1 = plsc.load_gather(rows, [rs, iota16]) * al
            a0 = plsc.load_gather(acc, [dsp, iota])
            a1 = plsc.load_gather(acc, [dsp, iota16])
            plsc.store_scatter(acc, [dsp, iota], a0 + r0)
            plsc.store_scatter(acc, [dsp, iota16], a1 + r1)
            return 0

        lax.fori_loop(0, nb, rb, 0)
        return 0

    lax.fori_loop(0, NCH, chunk, 0)

    b0 = biasv[pl.ds(0, 16)]
    b1 = biasv[pl.ds(16, 16)]

    def addb(r, _):
        rs = _splat_i(r)
        a0 = plsc.load_gather(acc, [rs, iota])
        a1 = plsc.load_gather(acc, [rs, iota16])
        plsc.store_scatter(acc, [rs, iota], a0 + b0)
        plsc.store_scatter(acc, [rs, iota16], a1 + b1)
        return 0

    lax.fori_loop(0, NLOC, addb, 0)
    pltpu.sync_copy(acc, hgat_hbm.at[pl.ds(n0, NLOC)])


def _k4_body(h_ref, pw_ref, q_ref, sc_ref):
    q = jnp.dot(h_ref[...], pw_ref[...], preferred_element_type=jnp.float32)
    q_ref[...] = q
    pw = pw_ref[...]
    nrm = jnp.sqrt(jnp.sum(pw * pw)) + 1e-16
    sc_ref[...] = 1.0 / (1.0 + jnp.exp(-(q / nrm)))


def _phase4(hgat, pool_w):
    return pl.pallas_call(
        _k4_body,
        grid=(NPAD // 512,),
        in_specs=[
            pl.BlockSpec((512, DIM1), lambda i: (i, 0)),
            pl.BlockSpec((DIM1, 1), lambda i: (0, 0)),
        ],
        out_specs=[
            pl.BlockSpec((512, 1), lambda i: (i, 0)),
            pl.BlockSpec((512, 1), lambda i: (i, 0)),
        ],
        out_shape=[
            jax.ShapeDtypeStruct((NPAD, 1), jnp.float32),
            jax.ShapeDtypeStruct((NPAD, 1), jnp.float32),
        ],
    )(hgat, pool_w.reshape(DIM1, 1))


@functools.partial(
    pl.kernel, mesh=_mesh,
    compiler_params=pltpu.CompilerParams(needs_layout_passes=False),
    out_type=[
        jax.ShapeDtypeStruct((NC, NGRAPH, 640), jnp.float32),
        jax.ShapeDtypeStruct((NC, NGRAPH, 640), jnp.int32),
        jax.ShapeDtypeStruct((NC, NGRAPH, DIM1), jnp.float32),
        jax.ShapeDtypeStruct((NC, NGRAPH, DIM1), jnp.float32),
    ],
    scratch_types=[
        pltpu.VMEM((640,), jnp.float32),
        pltpu.VMEM((640,), jnp.float32),
        pltpu.VMEM((640,), jnp.float32),
        pltpu.VMEM((640,), jnp.int32),
        pltpu.VMEM((384,), jnp.int32),
        pltpu.VMEM((384, 128), jnp.float32),
        pltpu.VMEM((DIM1,), jnp.float32),
        pltpu.VMEM((DIM1,), jnp.float32),
        pltpu.SemaphoreType.DMA,
    ],
)
def _sc5(q_hbm, sc_hbm, h_hbm,
         tvp_hbm, tip_hbm, gmp_hbm, gap_hbm,
         qv, scv, tvl, til, idxv, hrows, gmb, gab, sem):
    half = lax.axis_index("c")
    g = lax.axis_index("s")
    i0 = half * 320
    icnt = jnp.minimum(625 - i0, 320)
    pltpu.sync_copy(q_hbm.at[g], qv)
    pltpu.sync_copy(sc_hbm.at[g], scv)

    zf16 = jnp.zeros((16,), jnp.float32)
    zi16 = jnp.zeros((16,), jnp.int32)
    iota = lax.iota(jnp.int32, 16)

    def zb(j, _):
        tvl[pl.ds(j * 16, 16)] = zf16
        til[pl.ds(j * 16, 16)] = zi16
        return 0

    lax.fori_loop(0, 40, zb, 0)

    nbase = g * 625 + i0

    def ib(k, _):
        idxv[pl.ds(k * 16, 16)] = (nbase + k * 16) + iota
        return 0

    lax.fori_loop(0, 20, ib, 0)
    idxv[pl.ds(320, 16)] = jnp.broadcast_to(nbase, (16,)).astype(jnp.int32)
    idxv[pl.ds(336, 16)] = jnp.broadcast_to(nbase, (16,)).astype(jnp.int32)
    idxv[pl.ds(352, 16)] = jnp.broadcast_to(nbase, (16,)).astype(jnp.int32)
    idxv[pl.ds(368, 16)] = jnp.broadcast_to(nbase, (16,)).astype(jnp.int32)

    hs = []
    for ci in range(3):
        sl = pl.ds(ci * 128, 128)
        hs.append(pltpu.async_copy(h_hbm.at[idxv.at[sl]], hrows.at[sl], sem))
    for h in hs:
        h.wait()

    mask0 = iota == 0
    iota16 = iota + 16

    def rankb(il, carry):
        gm0, gm1, gs0, gs1 = carry
        ig = i0 + il
        vi = _sload(qv, ig)

        def jb(j, a):
            vj = qv[pl.ds(j * 16, 16)]
            jidx = j * 16 + iota
            cond = (vj > vi) | ((vj == vi) & (jidx < ig))
            return a + jnp.where(cond, 1, 0).astype(jnp.int32)

        accv = lax.fori_loop(0, 40, jb, zi16)
        rank = jnp.sum(accv)
        sel = rank < K
        si = _sload(scv, ig)

        @pl.when(sel)
        def _():
            _sstore(tvl, rank, si, mask0)
            _sstore(til, rank, ig, mask0)

        ils = _splat_i(il)
        r0 = plsc.load_gather(hrows, [ils, iota]) * si
        r1 = plsc.load_gather(hrows, [ils, iota16]) * si
        gm0 = jnp.where(sel, jnp.maximum(gm0, r0), gm0)
        gm1 = jnp.where(sel, jnp.maximum(gm1, r1), gm1)
        gs0 = jnp.where(sel, gs0 + r0, gs0)
        gs1 = jnp.where(sel, gs1 + r1, gs1)
        return (gm0, gm1, gs0, gs1)

    ninf16 = jnp.full((16,), -jnp.inf, jnp.float32)
    gm0, gm1, gs0, gs1 = lax.fori_loop(0, icnt, rankb, (ninf16, ninf16, zf16, zf16))
    gmb[pl.ds(0, 16)] = gm0
    gmb[pl.ds(16, 16)] = gm1
    gab[pl.ds(0, 16)] = gs0
    gab[pl.ds(16, 16)] = gs1
    pltpu.sync_copy(tvl, tvp_hbm.at[half, g])
    pltpu.sync_copy(til, tip_hbm.at[half, g])
    pltpu.sync_copy(gmb, gmp_hbm.at[half, g])
    pltpu.sync_copy(gab, gap_hbm.at[half, g])


def _k6_body(tvp, tip, gmp, gap, W1r, b1r, g1r, be1r, W2r, b2r, g2r, be2r,
             W3r, b3r, lo_ref, s2_ref, pm_ref):
    tv = tvp[0] + tvp[1]
    ti = tip[0] + tip[1]
    pm_ref[...] = ti + N_PER * jax.lax.broadcasted_iota(jnp.int32, (NGRAPH, 640), 0)
    s2_ref[...] = 1.0 / (1.0 + jnp.exp(-tv))
    gm = jnp.maximum(gmp[0], gmp[1])
    ga = (gap[0] + gap[1]) / float(K)
    z = jnp.concatenate([gm, ga], axis=1)
    z = jnp.maximum(jnp.dot(z, W1r[...], preferred_element_type=jnp.float32) + b1r[...], 0.0)
    z = g1r[...] * z + be1r[...]
    z = jnp.maximum(jnp.dot(z, W2r[...], preferred_element_type=jnp.float32) + b2r[...], 0.0)
    z = g2r[...] * z + be2r[...]
    u = jnp.dot(z, W3r[...], preferred_element_type=jnp.float32) + b3r[...]
    mx = jnp.max(u, axis=1, keepdims=True)
    sh = u - mx
    lo_ref[...] = sh - jnp.log(jnp.sum(jnp.exp(sh), axis=1, keepdims=True))


def _phase6(tvp, tip, gmp, gap, W1, b1, g1, be1, W2, b2, g2, be2, W3, b3):
    full = lambda s: pl.BlockSpec(s, lambda: tuple(0 for _ in s))
    return pl.pallas_call(
        _k6_body,
        in_specs=[
            full((NC, NGRAPH, 640)), full((NC, NGRAPH, 640)),
            full((NC, NGRAPH, DIM1)), full((NC, NGRAPH, DIM1)),
            full((2 * DIM1, DIM1)), full((1, DIM1)), full((1, DIM1)), full((1, DIM1)),
            full((DIM1, 8)), full((1, 8)), full((1, 8)), full((1, 8)),
            full((8, 2)), full((1, 2)),
        ],
        out_specs=[
            full((NGRAPH, 2)), full((NGRAPH, 640)), full((NGRAPH, 640)),
        ],
        out_shape=[
            jax.ShapeDtypeStruct((NGRAPH, 2), jnp.float32),
            jax.ShapeDtypeStruct((NGRAPH, 640), jnp.float32),
            jax.ShapeDtypeStruct((NGRAPH, 640), jnp.int32),
        ],
    )(tvp, tip, gmp, gap, W1, b1.reshape(1, -1), g1.reshape(1, -1), be1.reshape(1, -1),
      W2, b2.reshape(1, -1), g2.reshape(1, -1), be2.reshape(1, -1), W3, b3.reshape(1, -1))


def kernel(x, edge_index, batch, edge_attr, W_gat, att_src, att_dst, lin_edge,
           att_edge, bias_gat, pool_w, W1, b1, g1, be1, W2, b2, g2, be2, W3, b3):
    src = edge_index[0]
    dst = edge_index[1]

    h, a_s, a_d = _phase1(x, W_gat, att_src, att_dst)
    a_e = _phase_ae(edge_attr, lin_edge, att_edge).reshape(-1)

    e_arr, elist, counts = _sc2(src, dst, a_e, a_s.reshape(-1), a_d.reshape(-1))
    h128 = jnp.pad(h, ((0, 0), (0, 128 - DIM1)))
    hgat = _sc3(src, dst, e_arr, h128, elist, counts, bias_gat)

    q, sco = _phase4(hgat, pool_w)
    qg = q.reshape(-1)[:N].reshape(NGRAPH, N_PER)
    qg = jnp.pad(qg, ((0, 0), (0, 640 - N_PER)), constant_values=-jnp.inf)
    sg = sco.reshape(-1)[:N].reshape(NGRAPH, N_PER)
    sg = jnp.pad(sg, ((0, 0), (0, 640 - N_PER)))

    hg128 = jnp.pad(hgat, ((0, 0), (0, 128 - DIM1)))
    tvp, tip, gmp, gap = _sc5(qg, sg, hg128)
    logits, sig2, perm = _phase6(tvp, tip, gmp, gap,
                                 W1, b1, g1, be1, W2, b2, g2, be2, W3, b3)
    return (logits, pool_w.reshape(1, -1),
            sig2[:, :K], perm[:, :K])
